# baseline (speedup 1.0000x reference)
import functools

import jax
import jax.numpy as jnp
from jax import lax
from jax.experimental import pallas as pl
from jax.experimental.pallas import tpu as pltpu
from jax.experimental.pallas import tpu_sc as plsc

D = 512
B = 8192
K = 17
LANE = 16

NC, NS = 2, 16
NW = NC * NS
EPW = B // NW
CHUNK = 8
NCH = EPW // CHUNK
ROWS = CHUNK * K
XR = 72
YR = ROWS - XR
COLV = D // LANE

_mesh = plsc.VectorSubcoreMesh(
    core_axis_name="c", subcore_axis_name="s", num_cores=NC, num_subcores=NS
)


@functools.partial(
    pl.kernel,
    out_type=jax.ShapeDtypeStruct((B, D), jnp.float32),
    mesh=_mesh,
    scratch_types=[
        [pltpu.VMEM((ROWS,), jnp.int32) for _ in range(3)],
        [pltpu.VMEM((XR, D), jnp.float32) for _ in range(3)],
        [pltpu.VMEM((CHUNK, D), jnp.float32) for _ in range(3)],
        [pltpu.SemaphoreType.DMA for _ in range(3)],
        [pltpu.SemaphoreType.DMA for _ in range(3)],
        [pltpu.SemaphoreType.DMA for _ in range(3)],
    ],
)
def _gather_sum(
    feat_hbm, idx_hbm, out_hbm, ibufs, bufs, accs, sem_g, sem_i, sem_o
):
    wid = lax.axis_index("s") * NC + lax.axis_index("c")
    ibase = wid * EPW * K

    def idx_copy(c, t):
        return pltpu.make_async_copy(
            idx_hbm.at[pl.ds(ibase + c * ROWS, ROWS)], ibufs[t], sem_i[t]
        )

    def gather_x(t, bx):
        return pltpu.make_async_copy(
            feat_hbm.at[ibufs[t].at[pl.ds(0, XR)]], bufs[bx], sem_g[bx]
        )

    def gather_y(t, by):
        return pltpu.make_async_copy(
            feat_hbm.at[ibufs[t].at[pl.ds(XR, YR)]],
            bufs[by].at[pl.ds(0, YR)],
            sem_g[by],
        )

    def out_copy(c, t):
        return pltpu.make_async_copy(
            accs[t], out_hbm.at[pl.ds(wid * EPW + c * CHUNK, CHUNK)], sem_o[t]
        )

    def colx(xb, acc_v):
        def body(cv, c2):
            sl = pl.ds(cv * LANE, LANE)
            for j in range(4):
                base = j * K
                acc = xb[base, sl]
                for r in range(1, K):
                    acc = acc + xb[base + r, sl]
                acc_v[j, sl] = acc
            acc4 = xb[68, sl]
            for r in range(69, 72):
                acc4 = acc4 + xb[r, sl]
            acc_v[4, sl] = acc4
            return c2

        lax.fori_loop(0, COLV, body, 0, unroll=2)

    def coly(yb, acc_v):
        def body(cv, c2):
            sl = pl.ds(cv * LANE, LANE)
            acc4 = acc_v[4, sl]
            for r in range(13):
                acc4 = acc4 + yb[r, sl]
            acc_v[4, sl] = acc4
            for j in range(5, 8):
                base = j * K - XR
                acc = yb[base, sl]
                for r in range(1, K):
                    acc = acc + yb[base + r, sl]
                acc_v[j, sl] = acc
            return c2

        lax.fori_loop(0, COLV, body, 0, unroll=2)

    def chunk_step(c, t, last):
        bx = (2 * t) % 3
        by = (2 * t + 1) % 3
        t1 = (t + 1) % 3
        t2 = (t + 2) % 3

        @pl.when(c >= 3)
        def _drain_out():
            out_copy(c - 3, t).wait()

        if not last:
            idx_copy(c + 2, t2).start()

        gather_x(t, bx).wait()
        colx(bufs[bx], accs[t])

        if not last:
            gather_y(t1, bx).start()

        gather_y(t, by).wait()
        coly(bufs[by], accs[t])

        if not last:
            idx_copy(c + 2, t2).wait()
            gather_x(t2, by).start()

        out_copy(c, t).start()

    idx_copy(0, 0).start()
    idx_copy(0, 0).wait()
    gather_x(0, 0).start()
    idx_copy(1, 1).start()
    idx_copy(1, 1).wait()
    gather_y(0, 1).start()
    gather_x(1, 2).start()

    def super_body(s, carry):
        for t in range(3):
            chunk_step(s * 3 + t, t, last=False)
        return carry

    lax.fori_loop(0, (NCH - 2) // 3, super_body, 0)

    out_copy(NCH - 5, 0).wait()
    gather_x(0, 0).wait()
    colx(bufs[0], accs[0])
    gather_y(1, 0).start()
    gather_y(0, 1).wait()
    coly(bufs[1], accs[0])
    out_copy(NCH - 2, 0).start()

    out_copy(NCH - 4, 1).wait()
    gather_x(1, 2).wait()
    colx(bufs[2], accs[1])
    gather_y(1, 0).wait()
    coly(bufs[0], accs[1])
    out_copy(NCH - 1, 1).start()

    out_copy(NCH - 3, 2).wait()
    out_copy(NCH - 2, 0).wait()
    out_copy(NCH - 1, 1).wait()


BM = 1024


def _mm_body(x_ref, w_ref, o_ref):
    y = jnp.dot(x_ref[...], w_ref[...], preferred_element_type=jnp.float32)
    o_ref[...] = jnp.maximum(y * (1.0 / K), 0.0)


def _matmul_relu(x, w):
    return pl.pallas_call(
        _mm_body,
        grid=(B // BM,),
        in_specs=[
            pl.BlockSpec((BM, D), lambda i: (i, 0)),
            pl.BlockSpec((D, D), lambda i: (0, 0)),
        ],
        out_specs=pl.BlockSpec((BM, D), lambda i: (i, 0)),
        out_shape=jax.ShapeDtypeStruct((B, D), jnp.float32),
    )(x, w)


def kernel(features, node, neighbours, neigh_weights):
    idx = jnp.concatenate([neighbours, node], axis=1).reshape(-1)
    sums = _gather_sum(features, idx)
    return _matmul_relu(sums, neigh_weights)


# speedup vs baseline: 2.6574x; 2.6574x over previous
import functools

import jax
import jax.numpy as jnp
from jax import lax
from jax.experimental import pallas as pl
from jax.experimental.pallas import tpu as pltpu
from jax.experimental.pallas import tpu_sc as plsc

D = 512
B = 8192
K = 17
LANE = 16

NC, NS = 2, 16
NW = NC * NS
EPW = B // NW
CHUNK = 8
NCH = EPW // CHUNK
ROWS = CHUNK * K
XR = 72
YR = ROWS - XR
COLV = D // LANE

_mesh = plsc.VectorSubcoreMesh(
    core_axis_name="c", subcore_axis_name="s", num_cores=NC, num_subcores=NS
)


@functools.partial(
    pl.kernel,
    out_type=jax.ShapeDtypeStruct((B, D), jnp.float32),
    mesh=_mesh,
    scratch_types=[
        [pltpu.VMEM((ROWS,), jnp.int32) for _ in range(3)],
        [pltpu.VMEM((XR, D), jnp.float32) for _ in range(3)],
        [pltpu.VMEM((CHUNK, D), jnp.float32) for _ in range(3)],
        [pltpu.SemaphoreType.DMA for _ in range(3)],
        [pltpu.SemaphoreType.DMA for _ in range(3)],
        [pltpu.SemaphoreType.DMA for _ in range(3)],
    ],
)
def _gather_sum(
    feat_hbm, idx_hbm, out_hbm, ibufs, bufs, accs, sem_g, sem_i, sem_o
):
    wid = lax.axis_index("s") * NC + lax.axis_index("c")
    ibase = wid * EPW * K

    def idx_copy(c, t):
        return pltpu.make_async_copy(
            idx_hbm.at[pl.ds(ibase + c * ROWS, ROWS)], ibufs[t], sem_i[t]
        )

    def gather_x(t, bx):
        return pltpu.make_async_copy(
            feat_hbm.at[ibufs[t].at[pl.ds(0, XR)]], bufs[bx], sem_g[bx]
        )

    def gather_y(t, by):
        return pltpu.make_async_copy(
            feat_hbm.at[ibufs[t].at[pl.ds(XR, YR)]],
            bufs[by].at[pl.ds(0, YR)],
            sem_g[by],
        )

    def out_copy(c, t):
        return pltpu.make_async_copy(
            accs[t], out_hbm.at[pl.ds(wid * EPW + c * CHUNK, CHUNK)], sem_o[t]
        )

    def colx(xb, acc_v):
        def body(cv, c2):
            sl = pl.ds(cv * LANE, LANE)
            for j in range(4):
                base = j * K
                acc = xb[base, sl]
                for r in range(1, K):
                    acc = acc + xb[base + r, sl]
                acc_v[j, sl] = acc
            acc4 = xb[68, sl]
            for r in range(69, 72):
                acc4 = acc4 + xb[r, sl]
            acc_v[4, sl] = acc4
            return c2

        lax.fori_loop(0, COLV, body, 0)

    def coly(yb, acc_v):
        def body(cv, c2):
            sl = pl.ds(cv * LANE, LANE)
            acc4 = acc_v[4, sl]
            for r in range(13):
                acc4 = acc4 + yb[r, sl]
            acc_v[4, sl] = acc4
            for j in range(5, 8):
                base = j * K - XR
                acc = yb[base, sl]
                for r in range(1, K):
                    acc = acc + yb[base + r, sl]
                acc_v[j, sl] = acc
            return c2

        lax.fori_loop(0, COLV, body, 0)

    def chunk_step(c, t, last):
        bx = (2 * t) % 3
        by = (2 * t + 1) % 3
        t1 = (t + 1) % 3
        t2 = (t + 2) % 3

        @pl.when(c >= 3)
        def _drain_out():
            out_copy(c - 3, t).wait()

        if not last:
            idx_copy(c + 2, t2).start()

        gather_x(t, bx).wait()
        colx(bufs[bx], accs[t])

        if not last:
            gather_y(t1, bx).start()

        gather_y(t, by).wait()
        coly(bufs[by], accs[t])

        if not last:
            idx_copy(c + 2, t2).wait()
            gather_x(t2, by).start()

        out_copy(c, t).start()

    idx_copy(0, 0).start()
    idx_copy(0, 0).wait()
    gather_x(0, 0).start()
    idx_copy(1, 1).start()
    idx_copy(1, 1).wait()
    gather_y(0, 1).start()
    gather_x(1, 2).start()

    def super_body(s, carry):
        for t in range(3):
            chunk_step(s * 3 + t, t, last=False)
        return carry

    lax.fori_loop(0, (NCH - 2) // 3, super_body, 0)

    out_copy(NCH - 5, 0).wait()
    gather_x(0, 0).wait()
    colx(bufs[0], accs[0])
    gather_y(1, 0).start()
    gather_y(0, 1).wait()
    coly(bufs[1], accs[0])
    out_copy(NCH - 2, 0).start()

    out_copy(NCH - 4, 1).wait()
    gather_x(1, 2).wait()
    colx(bufs[2], accs[1])
    gather_y(1, 0).wait()
    coly(bufs[0], accs[1])
    out_copy(NCH - 1, 1).start()

    out_copy(NCH - 3, 2).wait()
    out_copy(NCH - 2, 0).wait()
    out_copy(NCH - 1, 1).wait()


BM = 2048


def _mm_body(x_ref, w_ref, o_ref):
    y = jnp.dot(x_ref[...], w_ref[...], preferred_element_type=jnp.float32)
    o_ref[...] = jnp.maximum(y * (1.0 / K), 0.0)


def _matmul_relu(x, w):
    return pl.pallas_call(
        _mm_body,
        grid=(B // BM,),
        in_specs=[
            pl.BlockSpec((BM, D), lambda i: (i, 0)),
            pl.BlockSpec((D, D), lambda i: (0, 0)),
        ],
        out_specs=pl.BlockSpec((BM, D), lambda i: (i, 0)),
        out_shape=jax.ShapeDtypeStruct((B, D), jnp.float32),
    )(x, w)


def kernel(features, node, neighbours, neigh_weights):
    idx = jnp.concatenate([neighbours, node], axis=1).reshape(-1)
    sums = _gather_sum(features, idx)
    return _matmul_relu(sums, neigh_weights)
